# Initial kernel scaffold; baseline (speedup 1.0000x reference)
import jax
import jax.numpy as jnp
from jax.experimental import pallas as pl


def kernel(x, sn, node, node_knn_I, k):
    raise NotImplementedError("write your pallas kernel here")


# speedup vs baseline: 67.9704x; 67.9704x over previous
import jax
import jax.numpy as jnp
from jax import lax
from jax.experimental import pallas as pl
from jax.experimental.pallas import tpu as pltpu
from jax.experimental.pallas import tpu_sc as plsc

NC = 2
NS = 16
L = 16

B = 8
N = 8192
M = 64
K = 3
SOM_K = 9

W = NC * NS
WPB = W // B
NP = N // WPB
NCHUNK = NP // L


def koff_ds(off):
    return pl.ds(off, NP)


def _body(x_hbm, sn_hbm, nodes_hbm, knnt_hbm,
          out_hbm, knn_hbm, mask_hbm,
          x0, x1, x2, sn0, sn1, sn2, node_v, knn_v, idx_v, hist_v, part_v,
          red0, red1, red2, red3,
          cm_v, knnc_v, mask_v,
          d0, d1_, d2_, d3_, d4, d5, d6, d7, d8,
          shared_part, shared_mean):
    xs = (x0, x1, x2)
    sns = (sn0, sn1, sn2)
    decs = (d0, d1_, d2_, d3_, d4, d5, d6, d7, d8)
    c = lax.axis_index("c")
    s = lax.axis_index("s")
    b = c * (B // NC) + s // WPB
    g = s // WPB
    q = s % WPB
    qoff = pl.multiple_of(q * NP, NP)

    for cc in range(3):
        pltpu.sync_copy(x_hbm.at[b * 3 + cc, :, pl.ds(qoff, NP)], xs[cc])
        pltpu.sync_copy(sn_hbm.at[b * 3 + cc, :, pl.ds(qoff, NP)], sns[cc])
    pltpu.sync_copy(nodes_hbm.at[b], node_v)
    pltpu.sync_copy(knnt_hbm.at[b], knn_v)

    lane = lax.broadcasted_iota(jnp.int32, (L,), 0)
    lane_base = lane * M

    def zero_body(i, _):
        hist_v[pl.ds(i * L, L)] = jnp.zeros((L,), jnp.float32)
        return 0
    lax.fori_loop(0, (4 * M * L) // L, zero_body, 0)

    inf = jnp.full((L,), jnp.inf, jnp.float32)
    zero_i = jnp.zeros((L,), jnp.int32)
    ones = jnp.full((L,), 1.0, jnp.float32)

    def chunk_body(t, _):
        po = t * L
        px = x0[0, pl.ds(po, L)]
        py = x1[0, pl.ds(po, L)]
        pz = x2[0, pl.ds(po, L)]

        def m_body(m, carry):
            d1, d2, d3, i1, i2, i3 = carry
            mo = m * L
            t0 = px - node_v[0, pl.ds(mo, L)]
            t1 = py - node_v[1, pl.ds(mo, L)]
            t2 = pz - node_v[2, pl.ds(mo, L)]
            d = t0 * t0 + t1 * t1 + t2 * t2
            mi = jnp.full((L,), m, jnp.int32)
            c1 = d < d1
            c2 = d < d2
            c3 = d < d3
            d3n = jnp.where(c3, jnp.where(c2, d2, d), d3)
            i3n = jnp.where(c3, jnp.where(c2, i2, mi), i3)
            d2n = jnp.where(c2, jnp.where(c1, d1, d), d2)
            i2n = jnp.where(c2, jnp.where(c1, i1, mi), i2)
            d1n = jnp.where(c1, d, d1)
            i1n = jnp.where(c1, mi, i1)
            return (d1n, d2n, d3n, i1n, i2n, i3n)

        _, _, _, i1, i2, i3 = lax.fori_loop(
            0, M, m_body, (inf, inf, inf, zero_i, zero_i, zero_i))

        idx_v[pl.ds(po, L)] = i1
        idx_v[pl.ds(NP + po, L)] = i2
        idx_v[pl.ds(2 * NP + po, L)] = i3
        for ik in (i1, i2, i3):
            bidx = ik + lane_base
            plsc.addupdate_scatter(hist_v, [bidx], px)
            plsc.addupdate_scatter(hist_v, [bidx + M * L], py)
            plsc.addupdate_scatter(hist_v, [bidx + 2 * M * L], pz)
            plsc.addupdate_scatter(hist_v, [bidx + 3 * M * L], ones)
        return 0

    lax.fori_loop(0, NCHUNK, chunk_body, 0)

    for r in range(4):
        for jm in range(M // L):
            tot = hist_v[pl.ds(r * M * L + jm * L, L)]
            for ln in range(1, L):
                tot = tot + hist_v[pl.ds(r * M * L + ln * M + jm * L, L)]
            part_v[pl.ds(r * M + jm * L, L)] = tot

    pltpu.sync_copy(part_v, shared_part.at[s])
    plsc.subcore_barrier()

    @pl.when(q == 0)
    def _leader():
        pltpu.sync_copy(shared_part.at[s], red0)
        pltpu.sync_copy(shared_part.at[s + 1], red1)
        pltpu.sync_copy(shared_part.at[s + 2], red2)
        pltpu.sync_copy(shared_part.at[s + 3], red3)
        for jm in range(4 * M // L):
            tot = (red0[pl.ds(jm * L, L)] + red1[pl.ds(jm * L, L)]
                   + red2[pl.ds(jm * L, L)] + red3[pl.ds(jm * L, L)])
            part_v[pl.ds(jm * L, L)] = tot
        for jm in range(M // L):
            cnt = part_v[pl.ds(3 * M + jm * L, L)]
            den = cnt + 1e-5
            for cc in range(3):
                cm_v[pl.ds(cc * M + jm * L, L)] = (
                    part_v[pl.ds(cc * M + jm * L, L)] / den)
            mask_v[0, pl.ds(jm * L, L)] = jnp.where(
                cnt > 0.5, jnp.full((L,), 1.0, jnp.float32),
                jnp.zeros((L,), jnp.float32))
        for cc in range(3):
            for jm in range(M // L):
                acc = jnp.zeros((L,), jnp.float32)
                for kk in range(SOM_K):
                    nidx = knn_v[kk, pl.ds(jm * L, L)]
                    acc = acc + plsc.load_gather(cm_v, [nidx + cc * M])
                knnc_v[cc, pl.ds(jm * L, L)] = acc / 9.0
        pltpu.sync_copy(mask_v, mask_hbm.at[b])
        pltpu.sync_copy(knnc_v, knn_hbm.at[b])
        pltpu.sync_copy(cm_v, shared_mean.at[g])

    plsc.subcore_barrier()

    @pl.when(q != 0)
    def _followers():
        pltpu.sync_copy(shared_mean.at[g], cm_v)

    def out_body(t, _):
        po = t * L
        i1 = idx_v[pl.ds(po, L)]
        i2 = idx_v[pl.ds(NP + po, L)]
        i3 = idx_v[pl.ds(2 * NP + po, L)]
        for cc in range(3):
            pxc = xs[cc][0, pl.ds(po, L)]
            for kk, ik in enumerate((i1, i2, i3)):
                gv = plsc.load_gather(cm_v, [ik + cc * M])
                decs[cc * 3 + kk][0, pl.ds(po, L)] = pxc - gv
        return 0

    lax.fori_loop(0, NCHUNK, out_body, 0)

    for cc in range(3):
        for kk in range(3):
            koff = pl.multiple_of(kk * N + q * NP, NP)
            pltpu.sync_copy(decs[cc * 3 + kk],
                            out_hbm.at[b * 6 + cc, :, koff_ds(koff)])
            pltpu.sync_copy(sns[cc],
                            out_hbm.at[b * 6 + 3 + cc, :, koff_ds(koff)])


def kernel(x, sn, node, node_knn_I, k):
    del k
    x_r = x.reshape(B * 3, 1, N)
    sn_r = sn.reshape(B * 3, 1, N)
    nodes_splat = jnp.broadcast_to(
        node[:, :, :, None], (B, 3, M, L)).reshape(B, 3, M * L)
    knnt = jnp.swapaxes(node_knn_I.astype(jnp.int32), 1, 2)

    mesh = plsc.VectorSubcoreMesh(core_axis_name="c", subcore_axis_name="s")
    out, knnc, mask = pl.kernel(
        _body,
        out_type=(
            jax.ShapeDtypeStruct((B * 6, 1, K * N), jnp.float32),
            jax.ShapeDtypeStruct((B, 3, M), jnp.float32),
            jax.ShapeDtypeStruct((B, 1, M), jnp.float32),
        ),
        mesh=mesh,
        compiler_params=pltpu.CompilerParams(needs_layout_passes=False),
        scratch_types=[
            pltpu.VMEM((1, NP), jnp.float32),
            pltpu.VMEM((1, NP), jnp.float32),
            pltpu.VMEM((1, NP), jnp.float32),
            pltpu.VMEM((1, NP), jnp.float32),
            pltpu.VMEM((1, NP), jnp.float32),
            pltpu.VMEM((1, NP), jnp.float32),
            pltpu.VMEM((3, M * L), jnp.float32),
            pltpu.VMEM((SOM_K, M), jnp.int32),
            pltpu.VMEM((3 * NP,), jnp.int32),
            pltpu.VMEM((4 * M * L,), jnp.float32),
            pltpu.VMEM((4 * M,), jnp.float32),
            pltpu.VMEM((4 * M,), jnp.float32),
            pltpu.VMEM((4 * M,), jnp.float32),
            pltpu.VMEM((4 * M,), jnp.float32),
            pltpu.VMEM((4 * M,), jnp.float32),
            pltpu.VMEM((2 * 128,), jnp.float32),
            pltpu.VMEM((3, M), jnp.float32),
            pltpu.VMEM((1, M), jnp.float32),
        ] + [pltpu.VMEM((1, NP), jnp.float32)] * 9 + [
            pltpu.VMEM_SHARED((NS, 4 * M), jnp.float32),
            pltpu.VMEM_SHARED((NS // WPB, 2 * 128), jnp.float32),
        ],
    )(x_r, sn_r, nodes_splat, knnt)
    return out.reshape(B, 6, K * N), knnc, mask.reshape(B, M)


# speedup vs baseline: 68.1659x; 1.0029x over previous
import jax
import jax.numpy as jnp
from jax import lax
from jax.experimental import pallas as pl
from jax.experimental.pallas import tpu as pltpu
from jax.experimental.pallas import tpu_sc as plsc

NC = 2
NS = 16
L = 16

B = 8
N = 8192
M = 64
K = 3
SOM_K = 9

W = NC * NS
WPB = W // B
NP = N // WPB
NCHUNK = NP // L


def koff_ds(off):
    return pl.ds(off, NP)


def _body(x_hbm, sn_hbm, nodes_hbm, knnt_hbm,
          out_hbm, knn_hbm, mask_hbm,
          x0, x1, x2, sn0, sn1, sn2, node_v, knn_v, idx_v, hist_v, part_v,
          red0, red1, red2, red3,
          cm_v, knnc_v, mask_v,
          d0, d1_, d2_, d3_, d4, d5, d6, d7, d8,
          dma_sem, shared_part, shared_mean):
    xs = (x0, x1, x2)
    sns = (sn0, sn1, sn2)
    decs = (d0, d1_, d2_, d3_, d4, d5, d6, d7, d8)
    c = lax.axis_index("c")
    s = lax.axis_index("s")
    b = c * (B // NC) + s // WPB
    g = s // WPB
    q = s % WPB
    qoff = pl.multiple_of(q * NP, NP)

    for cc in range(3):
        pltpu.sync_copy(x_hbm.at[b * 3 + cc, :, pl.ds(qoff, NP)], xs[cc])
        pltpu.sync_copy(sn_hbm.at[b * 3 + cc, :, pl.ds(qoff, NP)], sns[cc])
    pltpu.sync_copy(nodes_hbm.at[b], node_v)
    pltpu.sync_copy(knnt_hbm.at[b], knn_v)

    lane = lax.broadcasted_iota(jnp.int32, (L,), 0)
    lane_base = lane * M

    def zero_body(i, _):
        hist_v[pl.ds(i * L, L)] = jnp.zeros((L,), jnp.float32)
        return 0
    lax.fori_loop(0, (4 * M * L) // L, zero_body, 0)

    inf = jnp.full((L,), jnp.inf, jnp.float32)
    zero_i = jnp.zeros((L,), jnp.int32)
    ones = jnp.full((L,), 1.0, jnp.float32)

    def chunk_body(t, _):
        poa = t * (2 * L)
        pob = poa + L
        pa = tuple(xc[0, pl.ds(poa, L)] for xc in xs)
        pb = tuple(xc[0, pl.ds(pob, L)] for xc in xs)

        def insert(p, m, mi, carry):
            d1, d2, d3, i1, i2, i3 = carry
            mo = m * L
            t0 = p[0] - node_v[0, pl.ds(mo, L)]
            t1 = p[1] - node_v[1, pl.ds(mo, L)]
            t2 = p[2] - node_v[2, pl.ds(mo, L)]
            d = t0 * t0 + t1 * t1 + t2 * t2
            c1 = d < d1
            c2 = d < d2
            c3 = d < d3
            d3n = jnp.where(c3, jnp.where(c2, d2, d), d3)
            i3n = jnp.where(c3, jnp.where(c2, i2, mi), i3)
            d2n = jnp.where(c2, jnp.where(c1, d1, d), d2)
            i2n = jnp.where(c2, jnp.where(c1, i1, mi), i2)
            d1n = jnp.where(c1, d, d1)
            i1n = jnp.where(c1, mi, i1)
            return (d1n, d2n, d3n, i1n, i2n, i3n)

        def m_body(mh, carry):
            ca, cb = carry[:6], carry[6:]
            for u in range(2):
                m = mh * 2 + u
                mi = jnp.full((L,), m, jnp.int32)
                ca = insert(pa, m, mi, ca)
                cb = insert(pb, m, mi, cb)
            return ca + cb

        init = (inf, inf, inf, zero_i, zero_i, zero_i)
        res = lax.fori_loop(0, M // 2, m_body, init + init)
        ia = res[3:6]
        ib = res[9:12]

        for po, p, ii in ((poa, pa, ia), (pob, pb, ib)):
            idx_v[pl.ds(po, L)] = ii[0]
            idx_v[pl.ds(NP + po, L)] = ii[1]
            idx_v[pl.ds(2 * NP + po, L)] = ii[2]
            for ik in ii:
                bidx = ik + lane_base
                plsc.addupdate_scatter(hist_v, [bidx], p[0])
                plsc.addupdate_scatter(hist_v, [bidx + M * L], p[1])
                plsc.addupdate_scatter(hist_v, [bidx + 2 * M * L], p[2])
                plsc.addupdate_scatter(hist_v, [bidx + 3 * M * L], ones)
        return 0

    lax.fori_loop(0, NCHUNK // 2, chunk_body, 0)

    for r in range(4):
        for jm in range(M // L):
            tot = hist_v[pl.ds(r * M * L + jm * L, L)]
            for ln in range(1, L):
                tot = tot + hist_v[pl.ds(r * M * L + ln * M + jm * L, L)]
            part_v[pl.ds(r * M + jm * L, L)] = tot

    pltpu.sync_copy(part_v, shared_part.at[s])
    plsc.subcore_barrier()

    @pl.when(q == 0)
    def _leader():
        pltpu.sync_copy(shared_part.at[s], red0)
        pltpu.sync_copy(shared_part.at[s + 1], red1)
        pltpu.sync_copy(shared_part.at[s + 2], red2)
        pltpu.sync_copy(shared_part.at[s + 3], red3)
        for jm in range(4 * M // L):
            tot = (red0[pl.ds(jm * L, L)] + red1[pl.ds(jm * L, L)]
                   + red2[pl.ds(jm * L, L)] + red3[pl.ds(jm * L, L)])
            part_v[pl.ds(jm * L, L)] = tot
        for jm in range(M // L):
            cnt = part_v[pl.ds(3 * M + jm * L, L)]
            den = cnt + 1e-5
            for cc in range(3):
                cm_v[pl.ds(cc * M + jm * L, L)] = (
                    part_v[pl.ds(cc * M + jm * L, L)] / den)
            mask_v[0, pl.ds(jm * L, L)] = jnp.where(
                cnt > 0.5, jnp.full((L,), 1.0, jnp.float32),
                jnp.zeros((L,), jnp.float32))
        for cc in range(3):
            for jm in range(M // L):
                acc = jnp.zeros((L,), jnp.float32)
                for kk in range(SOM_K):
                    nidx = knn_v[kk, pl.ds(jm * L, L)]
                    acc = acc + plsc.load_gather(cm_v, [nidx + cc * M])
                knnc_v[cc, pl.ds(jm * L, L)] = acc / 9.0
        pltpu.sync_copy(mask_v, mask_hbm.at[b])
        pltpu.sync_copy(knnc_v, knn_hbm.at[b])
        pltpu.sync_copy(cm_v, shared_mean.at[g])

    plsc.subcore_barrier()

    @pl.when(q != 0)
    def _followers():
        pltpu.sync_copy(shared_mean.at[g], cm_v)

    def out_body(t, _):
        po = t * L
        i1 = idx_v[pl.ds(po, L)]
        i2 = idx_v[pl.ds(NP + po, L)]
        i3 = idx_v[pl.ds(2 * NP + po, L)]
        for cc in range(3):
            pxc = xs[cc][0, pl.ds(po, L)]
            for kk, ik in enumerate((i1, i2, i3)):
                gv = plsc.load_gather(cm_v, [ik + cc * M])
                decs[cc * 3 + kk][0, pl.ds(po, L)] = pxc - gv
        return 0

    lax.fori_loop(0, NCHUNK, out_body, 0)

    handles = []
    for cc in range(3):
        for kk in range(3):
            koff = pl.multiple_of(kk * N + q * NP, NP)
            handles.append(pltpu.async_copy(
                decs[cc * 3 + kk], out_hbm.at[b * 6 + cc, :, koff_ds(koff)],
                dma_sem))
            handles.append(pltpu.async_copy(
                sns[cc], out_hbm.at[b * 6 + 3 + cc, :, koff_ds(koff)],
                dma_sem))
    for h in handles:
        h.wait()


def kernel(x, sn, node, node_knn_I, k):
    del k
    x_r = x.reshape(B * 3, 1, N)
    sn_r = sn.reshape(B * 3, 1, N)
    nodes_splat = jnp.broadcast_to(
        node[:, :, :, None], (B, 3, M, L)).reshape(B, 3, M * L)
    knnt = jnp.swapaxes(node_knn_I.astype(jnp.int32), 1, 2)

    mesh = plsc.VectorSubcoreMesh(core_axis_name="c", subcore_axis_name="s")
    out, knnc, mask = pl.kernel(
        _body,
        out_type=(
            jax.ShapeDtypeStruct((B * 6, 1, K * N), jnp.float32),
            jax.ShapeDtypeStruct((B, 3, M), jnp.float32),
            jax.ShapeDtypeStruct((B, 1, M), jnp.float32),
        ),
        mesh=mesh,
        compiler_params=pltpu.CompilerParams(needs_layout_passes=False),
        scratch_types=[
            pltpu.VMEM((1, NP), jnp.float32),
            pltpu.VMEM((1, NP), jnp.float32),
            pltpu.VMEM((1, NP), jnp.float32),
            pltpu.VMEM((1, NP), jnp.float32),
            pltpu.VMEM((1, NP), jnp.float32),
            pltpu.VMEM((1, NP), jnp.float32),
            pltpu.VMEM((3, M * L), jnp.float32),
            pltpu.VMEM((SOM_K, M), jnp.int32),
            pltpu.VMEM((3 * NP,), jnp.int32),
            pltpu.VMEM((4 * M * L,), jnp.float32),
            pltpu.VMEM((4 * M,), jnp.float32),
            pltpu.VMEM((4 * M,), jnp.float32),
            pltpu.VMEM((4 * M,), jnp.float32),
            pltpu.VMEM((4 * M,), jnp.float32),
            pltpu.VMEM((4 * M,), jnp.float32),
            pltpu.VMEM((2 * 128,), jnp.float32),
            pltpu.VMEM((3, M), jnp.float32),
            pltpu.VMEM((1, M), jnp.float32),
        ] + [pltpu.VMEM((1, NP), jnp.float32)] * 9 + [
            pltpu.SemaphoreType.DMA,
            pltpu.VMEM_SHARED((NS, 4 * M), jnp.float32),
            pltpu.VMEM_SHARED((NS // WPB, 2 * 128), jnp.float32),
        ],
    )(x_r, sn_r, nodes_splat, knnt)
    return out.reshape(B, 6, K * N), knnc, mask.reshape(B, M)


# speedup vs baseline: 83.3570x; 1.2229x over previous
import jax
import jax.numpy as jnp
from jax import lax
from jax.experimental import pallas as pl
from jax.experimental.pallas import tpu as pltpu
from jax.experimental.pallas import tpu_sc as plsc

NC = 2
NS = 16
L = 16

B = 8
N = 8192
M = 64
K = 3
SOM_K = 9

W = NC * NS
WPB = W // B
NP = N // WPB
NCHUNK = NP // L


def _body(x_hbm, sn_hbm, nodes_hbm, knn_hbm_in,
          out_hbm, knn_hbm, mask_hbm,
          x_v, sn_v, knn_v, node_v, idx_v, hist_v,
          part_v, red0, red1, red2, red3,
          cm_v, knnc_v, mask_v,
          obuf, dma_sem, shared_part, shared_mean):
    c = lax.axis_index("c")
    s = lax.axis_index("s")
    b = c * (B // NC) + s // WPB
    g = s // WPB
    q = s % WPB
    qoff = pl.multiple_of(q * NP, NP)

    handles = [
        pltpu.async_copy(x_hbm.at[b, :, pl.ds(qoff, NP)], x_v, dma_sem),
        pltpu.async_copy(sn_hbm.at[b, :, pl.ds(qoff, NP)], sn_v, dma_sem),
        pltpu.async_copy(nodes_hbm.at[b], node_v, dma_sem),
        pltpu.async_copy(knn_hbm_in.at[b], knn_v, dma_sem),
    ]
    for h in handles:
        h.wait()

    lane = lax.broadcasted_iota(jnp.int32, (L,), 0)
    lane_base = lane * M

    def zero_body(i, _):
        hist_v[pl.ds(i * L, L)] = jnp.zeros((L,), jnp.float32)
        return 0
    lax.fori_loop(0, (4 * M * L) // L, zero_body, 0)

    inf = jnp.full((L,), jnp.inf, jnp.float32)
    zero_i = jnp.zeros((L,), jnp.int32)
    ones = jnp.full((L,), 1.0, jnp.float32)

    def chunk_body(t, _):
        poa = t * (2 * L)
        pob = poa + L
        pa = tuple(x_v[cc, pl.ds(poa, L)] for cc in range(3))
        pb = tuple(x_v[cc, pl.ds(pob, L)] for cc in range(3))

        def insert(p, m, mi, carry):
            d1, d2, d3, i1, i2, i3 = carry
            mo = m * L
            d = node_v[3, pl.ds(mo, L)] - p[0] * node_v[0, pl.ds(mo, L)]
            d = d - p[1] * node_v[1, pl.ds(mo, L)]
            d = d - p[2] * node_v[2, pl.ds(mo, L)]
            c1 = d < d1
            c2 = d < d2
            c3 = d < d3
            d3n = jnp.where(c3, jnp.where(c2, d2, d), d3)
            i3n = jnp.where(c3, jnp.where(c2, i2, mi), i3)
            d2n = jnp.where(c2, jnp.where(c1, d1, d), d2)
            i2n = jnp.where(c2, jnp.where(c1, i1, mi), i2)
            d1n = jnp.where(c1, d, d1)
            i1n = jnp.where(c1, mi, i1)
            return (d1n, d2n, d3n, i1n, i2n, i3n)

        def m_body(mh, carry):
            ca, cb = carry[:6], carry[6:]
            for u in range(2):
                m = mh * 2 + u
                mi = jnp.full((L,), m, jnp.int32)
                ca = insert(pa, m, mi, ca)
                cb = insert(pb, m, mi, cb)
            return ca + cb

        init = (inf, inf, inf, zero_i, zero_i, zero_i)
        res = lax.fori_loop(0, M // 2, m_body, init + init)
        ia = res[3:6]
        ib = res[9:12]

        for po, p, ii in ((poa, pa, ia), (pob, pb, ib)):
            idx_v[pl.ds(po, L)] = ii[0]
            idx_v[pl.ds(NP + po, L)] = ii[1]
            idx_v[pl.ds(2 * NP + po, L)] = ii[2]
            for ik in ii:
                bidx = ik + lane_base
                plsc.addupdate_scatter(hist_v, [bidx], p[0])
                plsc.addupdate_scatter(hist_v, [bidx + M * L], p[1])
                plsc.addupdate_scatter(hist_v, [bidx + 2 * M * L], p[2])
                plsc.addupdate_scatter(hist_v, [bidx + 3 * M * L], ones)
        return 0

    lax.fori_loop(0, NCHUNK // 2, chunk_body, 0)

    for r in range(4):
        for jm in range(M // L):
            tot = hist_v[pl.ds(r * M * L + jm * L, L)]
            for ln in range(1, L):
                tot = tot + hist_v[pl.ds(r * M * L + ln * M + jm * L, L)]
            part_v[pl.ds(r * M + jm * L, L)] = tot

    pltpu.sync_copy(part_v, shared_part.at[s])
    plsc.subcore_barrier()

    @pl.when(q == 0)
    def _leader():
        pltpu.sync_copy(shared_part.at[s], red0)
        pltpu.sync_copy(shared_part.at[s + 1], red1)
        pltpu.sync_copy(shared_part.at[s + 2], red2)
        pltpu.sync_copy(shared_part.at[s + 3], red3)
        for jm in range(4 * M // L):
            tot = (red0[pl.ds(jm * L, L)] + red1[pl.ds(jm * L, L)]
                   + red2[pl.ds(jm * L, L)] + red3[pl.ds(jm * L, L)])
            part_v[pl.ds(jm * L, L)] = tot
        for jm in range(M // L):
            cnt = part_v[pl.ds(3 * M + jm * L, L)]
            den = cnt + 1e-5
            for cc in range(3):
                cm_v[pl.ds(cc * M + jm * L, L)] = (
                    part_v[pl.ds(cc * M + jm * L, L)] / den)
            mask_v[0, pl.ds(jm * L, L)] = jnp.where(
                cnt > 0.5, jnp.full((L,), 1.0, jnp.float32),
                jnp.zeros((L,), jnp.float32))
        for jm in range(M // L):
            acc = [jnp.zeros((L,), jnp.float32) for _ in range(3)]
            for kk in range(SOM_K):
                nidx = knn_v[kk, pl.ds(jm * L, L)]
                for cc in range(3):
                    acc[cc] = acc[cc] + plsc.load_gather(
                        cm_v, [nidx + cc * M])
            for cc in range(3):
                knnc_v[cc, pl.ds(jm * L, L)] = acc[cc] / 9.0
        pltpu.sync_copy(mask_v, mask_hbm.at[b])
        pltpu.sync_copy(knnc_v, knn_hbm.at[b])
        pltpu.sync_copy(cm_v, shared_mean.at[g])

    plsc.subcore_barrier()

    @pl.when(q != 0)
    def _followers():
        pltpu.sync_copy(shared_mean.at[g], cm_v)

    for kk in range(3):
        def out_body(t, _, kk=kk):
            po = t * L
            iv = idx_v[pl.ds(kk * NP + po, L)]
            for cc in range(3):
                gv = plsc.load_gather(cm_v, [iv + cc * M])
                obuf[cc, pl.ds(po, L)] = x_v[cc, pl.ds(po, L)] - gv
                if kk == 0:
                    obuf[3 + cc, pl.ds(po, L)] = sn_v[cc, pl.ds(po, L)]
            return 0

        lax.fori_loop(0, NCHUNK, out_body, 0)
        koff = pl.multiple_of(kk * N + q * NP, NP)
        pltpu.sync_copy(obuf, out_hbm.at[b, :, pl.ds(koff, NP)])


def kernel(x, sn, node, node_knn_I, k):
    del k
    node_pack = jnp.concatenate(
        [node + node, jnp.sum(node * node, axis=1, keepdims=True)], axis=1)
    nodes_splat = jnp.broadcast_to(
        node_pack[:, :, :, None], (B, 4, M, L)).reshape(B, 4, M * L)
    knnt = jnp.swapaxes(node_knn_I.astype(jnp.int32), 1, 2)

    mesh = plsc.VectorSubcoreMesh(core_axis_name="c", subcore_axis_name="s")
    out, knnc, mask = pl.kernel(
        _body,
        out_type=(
            jax.ShapeDtypeStruct((B, 6, K * N), jnp.float32),
            jax.ShapeDtypeStruct((B, 3, M), jnp.float32),
            jax.ShapeDtypeStruct((B, 1, M), jnp.float32),
        ),
        mesh=mesh,
        compiler_params=pltpu.CompilerParams(needs_layout_passes=False),
        scratch_types=[
            pltpu.VMEM((3, NP), jnp.float32),
            pltpu.VMEM((3, NP), jnp.float32),
            pltpu.VMEM((SOM_K, M), jnp.int32),
            pltpu.VMEM((4, M * L), jnp.float32),
            pltpu.VMEM((3 * NP,), jnp.int32),
            pltpu.VMEM((4 * M * L,), jnp.float32),
            pltpu.VMEM((4 * M,), jnp.float32),
            pltpu.VMEM((4 * M,), jnp.float32),
            pltpu.VMEM((4 * M,), jnp.float32),
            pltpu.VMEM((4 * M,), jnp.float32),
            pltpu.VMEM((4 * M,), jnp.float32),
            pltpu.VMEM((2 * 128,), jnp.float32),
            pltpu.VMEM((3, M), jnp.float32),
            pltpu.VMEM((1, M), jnp.float32),
            pltpu.VMEM((6, NP), jnp.float32),
            pltpu.SemaphoreType.DMA,
            pltpu.VMEM_SHARED((NS, 4 * M), jnp.float32),
            pltpu.VMEM_SHARED((NS // WPB, 2 * 128), jnp.float32),
        ],
    )(x, sn, nodes_splat, knnt)
    return out, knnc, mask.reshape(B, M)
